# Initial kernel scaffold; baseline (speedup 1.0000x reference)
import jax
import jax.numpy as jnp
from jax.experimental import pallas as pl


def kernel(w1, b1, w2, b2, wfc, bfc, p0, c2, x_nchw):
    raise NotImplementedError("write your pallas kernel here")


# speedup vs baseline: 1.4075x; 1.4075x over previous
import jax
import jax.numpy as jnp
from jax.experimental import pallas as pl
from jax.experimental.pallas import tpu as pltpu

_TB = 32
_QR = 152
_LQ = _TB * _QR
_LR = _LQ + 8
_M16 = _LQ + 32


def _feat_kernel(x_ref, w1q_ref, b1q_ref, w2_ref, b2_ref, o_ref):
    f32 = jnp.float32
    bf16 = jnp.bfloat16

    flat = jnp.concatenate(
        [x_ref[...], jnp.zeros((48, 64), bf16)], axis=0)
    sh = flat[1:_M16 + 1]

    x16 = jnp.concatenate(
        [flat[0:_M16]] +
        [jnp.concatenate([flat[0:_M16, 16 * kw:], sh[:, :16 * kw]], axis=1)
         for kw in (1, 2, 3)], axis=1)

    t = jnp.dot(x16, w1q_ref[...], preferred_element_type=f32)

    acc = (t[0:_LR, 0:128] + t[8:8 + _LR, 128:256] +
           t[16:16 + _LR, 256:384] + t[24:24 + _LR, 384:512])
    r1 = jnp.maximum(acc + b1q_ref[...], 0.0)

    mx = jnp.concatenate(
        [jnp.maximum(r1[:, 0:32], r1[:, 32:64]),
         jnp.maximum(r1[:, 64:96], r1[:, 96:128])], axis=1)
    my = jnp.maximum(mx[0:_LQ], mx[8:_LQ + 8])

    myr = my.reshape(_TB, 19, 8, 64)[:, 0:16]
    myr = myr.reshape(_TB, 8, 2, 8, 64)[:, :, 0]
    mb = myr.astype(bf16)
    ea = mb[:, :, :, 0:32]
    ob = mb[:, :, :, 32:64]

    w2 = w2_ref[...]
    pe = jnp.concatenate(
        [(ea if kw % 2 == 0 else ob)[:, kh: kh + 5, kw // 2: kw // 2 + 3, :]
         for kh in range(4) for kw in range(4)], axis=3)
    po = jnp.concatenate(
        [(ob if kw % 2 == 0 else ea)[:, kh: kh + 5,
                                     (kw + 1) // 2: (kw + 1) // 2 + 2, :]
         for kh in range(4) for kw in range(4)], axis=3)
    te = jnp.dot(pe.reshape(_TB * 15, 512), w2,
                 preferred_element_type=f32).reshape(_TB, 5, 3, 64)
    to = jnp.dot(po.reshape(_TB * 10, 512), w2,
                 preferred_element_type=f32).reshape(_TB, 5, 2, 64)
    h2 = jnp.concatenate(
        [te[:, :, 0:1], to[:, :, 0:1], te[:, :, 1:2],
         to[:, :, 1:2], te[:, :, 2:3]], axis=2)
    h2 = jnp.maximum(h2 + b2_ref[...], 0.0)
    o_ref[...] = h2.astype(bf16).reshape(_TB * 25, 64)


def _features(xflat, w1q, b1q, w2, b2):
    n = xflat.shape[0] // _QR
    return pl.pallas_call(
        _feat_kernel,
        out_shape=jax.ShapeDtypeStruct((n * 25, 64), jnp.bfloat16),
        grid=(n // _TB,),
        in_specs=[
            pl.BlockSpec((_LQ, 64), lambda i: (i, 0)),
            pl.BlockSpec((256, 512), lambda i: (0, 0)),
            pl.BlockSpec((1, 128), lambda i: (0, 0)),
            pl.BlockSpec((512, 64), lambda i: (0, 0)),
            pl.BlockSpec((1, 64), lambda i: (0, 0)),
        ],
        out_specs=pl.BlockSpec((_TB * 25, 64), lambda i: (i, 0)),
        compiler_params=pltpu.CompilerParams(
            dimension_semantics=("parallel",)),
    )(xflat, w1q, b1q, w2, b2)


def _fc_kernel(x_ref, w_ref, b_ref, o_ref):
    o_ref[...] = (jnp.dot(x_ref[...], w_ref[...],
                          preferred_element_type=jnp.float32) + b_ref[...])


def _fc(feat, wfc, bfc, tile_m=512):
    m, k = feat.shape
    _, n_out = wfc.shape
    while m % tile_m:
        tile_m //= 2
    return pl.pallas_call(
        _fc_kernel,
        out_shape=jax.ShapeDtypeStruct((m, n_out), jnp.float32),
        grid=(m // tile_m,),
        in_specs=[
            pl.BlockSpec((tile_m, k), lambda i: (i, 0)),
            pl.BlockSpec((k, n_out), lambda i: (0, 0)),
            pl.BlockSpec((1, n_out), lambda i: (0, 0)),
        ],
        out_specs=pl.BlockSpec((tile_m, n_out), lambda i: (i, 0)),
        compiler_params=pltpu.CompilerParams(
            dimension_semantics=("parallel",)),
    )(feat, wfc, bfc)


def kernel(w1, b1, w2, b2, wfc, bfc, p0, c2, x_nchw):
    del p0, c2
    n = x_nchw.shape[0]
    n_pad = -(-n // _TB) * _TB
    x = jnp.transpose(x_nchw, (0, 2, 3, 1))
    x = jnp.pad(x, ((0, n_pad - n), (0, 0), (0, 13), (0, 2)))
    x = x.astype(jnp.bfloat16).reshape(n_pad * _QR, 64)

    eye4 = jnp.eye(4, dtype=jnp.bfloat16)
    w1q = jnp.einsum('hwic,pq->wpihqc', w1.reshape(4, 4, 16, 32),
                     eye4).reshape(256, 512)
    b1q = jnp.tile(b1, (1, 4))

    feat = _features(x, w1q, b1q, w2, b2).reshape(n_pad, 1600)
    out = _fc(feat, wfc, bfc)
    return out[:n, :1444].reshape(n, 19, 19, 4)
